# Initial kernel scaffold; baseline (speedup 1.0000x reference)
import jax
import jax.numpy as jnp
from jax.experimental import pallas as pl


def kernel(tgt_chars, tgt_compos, ctx_words, noise, word_emb, char_emb, compo_emb):
    raise NotImplementedError("write your pallas kernel here")


# speedup vs baseline: 2.7289x; 2.7289x over previous
import functools

import jax
import jax.numpy as jnp
from jax import lax
from jax.experimental import pallas as pl
from jax.experimental.pallas import tpu as pltpu
from jax.experimental.pallas import tpu_sc as plsc

_B = 256
_W = 4
_NCH = 4
_NCO = 3
_D = 300
_NC = 2
_NS = 16
_NW = _NC * _NS
_BPW = _B // _NW
_L = 16
_NFULL = _D // _L
_TAILO = _NFULL * _L
_TAIL = _D - _TAILO

_GRID = 8
_NBLK = (_B * _W) // _GRID


def _sc_body(cidx_hbm, chidx_hbm, widx_hbm, compo_hbm, char_hbm, word_hbm,
             tgt_out, wctx_out,
             cidx_v, chidx_v, widx_v,
             crows_v, chrows_v, wrows_v, tacc_v,
             csem, hsem, wsem):
    wid = lax.axis_index("s") * _NC + lax.axis_index("c")
    nco = _BPW * _NCO
    nch = _BPW * _NCH
    nw = _BPW * _W

    pltpu.sync_copy(cidx_hbm.at[pl.ds(wid * nco, nco)], cidx_v)
    pltpu.sync_copy(chidx_hbm.at[pl.ds(wid * nch, nch)], chidx_v)
    pltpu.sync_copy(widx_hbm.at[pl.ds(wid * nw, nw)], widx_v)

    def _scalars(ref, n):
        vals = [None] * n
        starts = sorted({*range(0, n - _L + 1, _L), n - _L})
        for s in starts:
            v = ref[pl.ds(s, _L)]
            for l in range(_L):
                if vals[s + l] is None:
                    vals[s + l] = v[l]
        return vals

    cids = _scalars(cidx_v, nco)
    hids = _scalars(chidx_v, nch)
    wids = _scalars(widx_v, nw)

    cd = [pltpu.async_copy(compo_hbm.at[pl.ds(cids[r], 1)],
                           crows_v.at[pl.ds(r, 1)], csem)
          for r in range(nco)]
    hd = [pltpu.async_copy(char_hbm.at[pl.ds(hids[r], 1)],
                           chrows_v.at[pl.ds(r, 1)], hsem)
          for r in range(nch)]
    wd = [pltpu.async_copy(word_hbm.at[pl.ds(wids[r], 1)],
                           wrows_v.at[pl.ds(r, 1)], wsem)
          for r in range(nw)]

    def _cmask(j):
        return jnp.where(cids[j] != 1, 1.0, 0.0)

    def _hmask(j):
        return jnp.where(hids[j] != 1, 1.0, 0.0)

    for d in cd:
        d.wait()
    for d in hd:
        d.wait()

    offs = [k * _L for k in range(_NFULL)] + [_D - _L]

    def _accum_row(acc, rows_v, r, m):
        for k, o in enumerate(offs):
            acc[k] = acc[k] + rows_v[r, pl.ds(o, _L)] * m

    for b in range(_BPW):
        acc = [jnp.zeros((_L,), jnp.float32) for _ in range(len(offs))]
        for j in range(_NCO):
            _accum_row(acc, crows_v, b * _NCO + j, _cmask(b * _NCO + j))
        for j in range(_NCH):
            _accum_row(acc, chrows_v, b * _NCH + j, _hmask(b * _NCH + j))
        for k, o in enumerate(offs):
            tacc_v[pl.ds(b * _D + o, _L)] = acc[k]

    pltpu.sync_copy(tacc_v, tgt_out.at[pl.ds(wid * _BPW * _D, _BPW * _D)])

    for d in wd:
        d.wait()
    pltpu.sync_copy(wrows_v, wctx_out.at[pl.ds(wid * nw, nw)])


@functools.lru_cache(maxsize=1)
def _get_sc_gather():
    return functools.partial(
        pl.kernel,
        out_type=(jax.ShapeDtypeStruct((_B * _D,), jnp.float32),
                  jax.ShapeDtypeStruct((_B * _W, _D), jnp.float32)),
        mesh=plsc.VectorSubcoreMesh(core_axis_name="c", subcore_axis_name="s"),
        scratch_types=[
            pltpu.VMEM((_BPW * _NCO,), jnp.int32),
            pltpu.VMEM((_BPW * _NCH,), jnp.int32),
            pltpu.VMEM((_BPW * _W,), jnp.int32),
            pltpu.VMEM((_BPW * _NCO, _D), jnp.float32),
            pltpu.VMEM((_BPW * _NCH, _D), jnp.float32),
            pltpu.VMEM((_BPW * _W, _D), jnp.float32),
            pltpu.VMEM((_BPW * _D,), jnp.float32),
            pltpu.SemaphoreType.DMA,
            pltpu.SemaphoreType.DMA,
            pltpu.SemaphoreType.DMA,
        ],
    )(_sc_body)


def _tc_body(cw_ref, tgt_ref, wctx_ref, noise_ref, out_ref):
    i = pl.program_id(0)
    tgt = tgt_ref[...]
    nf = noise_ref[...].astype(jnp.float32)
    s = -lax.dot_general(tgt, nf, (((1,), (1,)), ((), ())),
                         preferred_element_type=jnp.float32,
                         precision=lax.Precision.HIGHEST)
    sig = 1.0 / (1.0 + jnp.exp(-s))
    part = jnp.sum(jnp.log(sig + 1e-32))

    @pl.when(i == 0)
    def _init():
        wctx = wctx_ref[...].reshape(_B, _W, _D)
        dots = jnp.sum(tgt[:, None, :] * wctx, axis=2)
        mask = (cw_ref[...] != 1).astype(jnp.float32)
        dots = dots * mask
        sd = 1.0 / (1.0 + jnp.exp(-dots))
        out_ref[...] = jnp.sum(jnp.log(sd)).reshape(1, 1)

    out_ref[...] = out_ref[...] + part

    @pl.when(i == _GRID - 1)
    def _fin():
        out_ref[...] = out_ref[...] * (-1.0 / _B)


_tc_loss = pl.pallas_call(
    _tc_body,
    grid=(_GRID,),
    in_specs=[
        pl.BlockSpec((_B, _W), lambda i: (0, 0)),
        pl.BlockSpec((_B, _D), lambda i: (0, 0)),
        pl.BlockSpec((_B * _W, _D), lambda i: (0, 0)),
        pl.BlockSpec((_NBLK, _D), lambda i: (i, 0)),
    ],
    out_specs=pl.BlockSpec((1, 1), lambda i: (0, 0)),
    out_shape=jax.ShapeDtypeStruct((1, 1), jnp.float32),
)


def kernel(tgt_chars, tgt_compos, ctx_words, noise, word_emb, char_emb,
           compo_emb):
    cidx = tgt_compos.reshape(-1).astype(jnp.int32)
    chidx = tgt_chars.reshape(-1).astype(jnp.int32)
    widx = ctx_words.reshape(-1).astype(jnp.int32)
    tgt_flat, wctx = _get_sc_gather()(cidx, chidx, widx, compo_emb, char_emb,
                                      word_emb)
    tgt = tgt_flat.reshape(_B, _D)
    noise2 = noise.reshape(_B * _W, _D).astype(jnp.int32)
    loss2d = _tc_loss(ctx_words.astype(jnp.int32), tgt, wctx, noise2)
    return loss2d[0, 0]


# speedup vs baseline: 2.8353x; 1.0390x over previous
import functools

import jax
import jax.numpy as jnp
from jax import lax
from jax.experimental import pallas as pl
from jax.experimental.pallas import tpu as pltpu
from jax.experimental.pallas import tpu_sc as plsc

_B = 256
_W = 4
_NCH = 4
_NCO = 3
_D = 300
_NC = 2
_NS = 16
_NW = _NC * _NS
_BPW = _B // _NW
_L = 16
_NFULL = _D // _L
_TAILO = _NFULL * _L
_TAIL = _D - _TAILO

_GRID = 8
_NBLK = (_B * _W) // _GRID


def _sc_body(cidx_hbm, chidx_hbm, widx_hbm, compo_hbm, char_hbm, word_hbm,
             tgt_out, dots_out,
             cidx_v, chidx_v, widx_v,
             crows_v, chrows_v, wrows_v, tacc_v, dout_v,
             csem, hsem, wsem):
    wid = lax.axis_index("s") * _NC + lax.axis_index("c")
    nco = _BPW * _NCO
    nch = _BPW * _NCH
    nw = _BPW * _W

    pltpu.sync_copy(cidx_hbm.at[pl.ds(wid * nco, nco)], cidx_v)
    pltpu.sync_copy(chidx_hbm.at[pl.ds(wid * nch, nch)], chidx_v)
    pltpu.sync_copy(widx_hbm.at[pl.ds(wid * nw, nw)], widx_v)

    def _scalars(ref, n):
        vals = [None] * n
        starts = sorted({*range(0, n - _L + 1, _L), n - _L})
        for s in starts:
            v = ref[pl.ds(s, _L)]
            for l in range(_L):
                if vals[s + l] is None:
                    vals[s + l] = v[l]
        return vals

    cids = _scalars(cidx_v, nco)
    hids = _scalars(chidx_v, nch)
    wids = _scalars(widx_v, nw)

    cd = [pltpu.async_copy(compo_hbm.at[pl.ds(cids[r], 1)],
                           crows_v.at[pl.ds(r, 1)], csem)
          for r in range(nco)]
    hd = [pltpu.async_copy(char_hbm.at[pl.ds(hids[r], 1)],
                           chrows_v.at[pl.ds(r, 1)], hsem)
          for r in range(nch)]
    wd = [pltpu.async_copy(word_hbm.at[pl.ds(wids[r], 1)],
                           wrows_v.at[pl.ds(r, 1)], wsem)
          for r in range(nw)]

    def _cmask(j):
        return jnp.where(cids[j] != 1, 1.0, 0.0)

    def _hmask(j):
        return jnp.where(hids[j] != 1, 1.0, 0.0)

    for d in cd:
        d.wait()
    for d in hd:
        d.wait()

    offs = [k * _L for k in range(_NFULL)] + [_D - _L]
    lanes = lax.iota(jnp.int32, _L)
    dmask = jnp.where(lanes >= (_L - _TAIL), 1.0, 0.0)

    def _accum_row(acc, rows_v, r, m):
        for k, o in enumerate(offs):
            acc[k] = acc[k] + rows_v[r, pl.ds(o, _L)] * m

    for d in wd:
        d.wait()

    for b in range(_BPW):
        acc = [jnp.zeros((_L,), jnp.float32) for _ in range(len(offs))]
        for j in range(_NCO):
            _accum_row(acc, crows_v, b * _NCO + j, _cmask(b * _NCO + j))
        for j in range(_NCH):
            _accum_row(acc, chrows_v, b * _NCH + j, _hmask(b * _NCH + j))
        for k, o in enumerate(offs):
            tacc_v[b, pl.ds(o, _L)] = acc[k]
        for w in range(_W):
            r = b * _W + w
            pvec = acc[0] * wrows_v[r, pl.ds(0, _L)]
            for k in range(1, _NFULL):
                pvec = pvec + acc[k] * wrows_v[r, pl.ds(offs[k], _L)]
            pvec = pvec + acc[_NFULL] * wrows_v[r, pl.ds(offs[_NFULL], _L)] * dmask
            dout_v[b, pl.ds(w * _L, _L)] = pvec

    pltpu.sync_copy(tacc_v, tgt_out.at[pl.ds(wid * _BPW, _BPW)])
    pltpu.sync_copy(dout_v, dots_out.at[pl.ds(wid * _BPW, _BPW)])


@functools.lru_cache(maxsize=1)
def _get_sc_gather():
    return functools.partial(
        pl.kernel,
        out_type=(jax.ShapeDtypeStruct((_B, _D), jnp.float32),
                  jax.ShapeDtypeStruct((_B, _W * _L), jnp.float32)),
        mesh=plsc.VectorSubcoreMesh(core_axis_name="c", subcore_axis_name="s"),
        scratch_types=[
            pltpu.VMEM((_BPW * _NCO,), jnp.int32),
            pltpu.VMEM((_BPW * _NCH,), jnp.int32),
            pltpu.VMEM((_BPW * _W,), jnp.int32),
            pltpu.VMEM((_BPW * _NCO, _D), jnp.float32),
            pltpu.VMEM((_BPW * _NCH, _D), jnp.float32),
            pltpu.VMEM((_BPW * _W, _D), jnp.float32),
            pltpu.VMEM((_BPW, _D), jnp.float32),
            pltpu.VMEM((_BPW, _W * _L), jnp.float32),
            pltpu.SemaphoreType.DMA,
            pltpu.SemaphoreType.DMA,
            pltpu.SemaphoreType.DMA,
        ],
    )(_sc_body)


def _tc_body(cw_ref, tgt_ref, dp_ref, noise_ref, out_ref):
    i = pl.program_id(0)
    tgt = tgt_ref[...]
    nf = noise_ref[...].astype(jnp.float32)
    s = -lax.dot_general(tgt, nf, (((1,), (1,)), ((), ())),
                         preferred_element_type=jnp.float32,
                         precision=lax.Precision.HIGHEST)
    sig = 1.0 / (1.0 + jnp.exp(-s))
    part = jnp.sum(jnp.log(sig + 1e-32))

    @pl.when(i == 0)
    def _init():
        gsel = (lax.broadcasted_iota(jnp.int32, (_W * _L, _W), 0) // _L ==
                lax.broadcasted_iota(jnp.int32, (_W * _L, _W), 1)
                ).astype(jnp.float32)
        dots = lax.dot_general(dp_ref[...], gsel, (((1,), (0,)), ((), ())),
                               preferred_element_type=jnp.float32,
                               precision=lax.Precision.HIGHEST)
        mask = (cw_ref[...] != 1).astype(jnp.float32)
        dots = dots * mask
        sd = 1.0 / (1.0 + jnp.exp(-dots))
        out_ref[...] = jnp.sum(jnp.log(sd)).reshape(1, 1)

    out_ref[...] = out_ref[...] + part

    @pl.when(i == _GRID - 1)
    def _fin():
        out_ref[...] = out_ref[...] * (-1.0 / _B)


_tc_loss = pl.pallas_call(
    _tc_body,
    grid=(_GRID,),
    in_specs=[
        pl.BlockSpec((_B, _W), lambda i: (0, 0)),
        pl.BlockSpec((_B, _D), lambda i: (0, 0)),
        pl.BlockSpec((_B, _W * _L), lambda i: (0, 0)),
        pl.BlockSpec((_NBLK, _D), lambda i: (i, 0)),
    ],
    out_specs=pl.BlockSpec((1, 1), lambda i: (0, 0)),
    out_shape=jax.ShapeDtypeStruct((1, 1), jnp.float32),
)


def kernel(tgt_chars, tgt_compos, ctx_words, noise, word_emb, char_emb,
           compo_emb):
    cidx = tgt_compos.reshape(-1).astype(jnp.int32)
    chidx = tgt_chars.reshape(-1).astype(jnp.int32)
    widx = ctx_words.reshape(-1).astype(jnp.int32)
    tgt, dpart = _get_sc_gather()(cidx, chidx, widx, compo_emb, char_emb,
                                  word_emb)
    noise2 = noise.reshape(_B * _W, _D).astype(jnp.int32)
    loss2d = _tc_loss(ctx_words.astype(jnp.int32), tgt, dpart, noise2)
    return loss2d[0, 0]


# speedup vs baseline: 3.1529x; 1.1120x over previous
import functools

import jax
import jax.numpy as jnp
from jax import lax
from jax.experimental import pallas as pl
from jax.experimental.pallas import tpu as pltpu
from jax.experimental.pallas import tpu_sc as plsc

_B = 256
_W = 4
_NCH = 4
_NCO = 3
_D = 300
_NWORD = 2010
_NCOMPO = 20000
_NC = 2
_NS = 16
_NW = _NC * _NS
_BPW = _B // _NW
_L = 16
_NFULL = _D // _L
_TAIL = _D - _NFULL * _L

_VB = 2048
_NA = -(-_NCOMPO // _VB)
_NB = 8
_NBLK = (_B * _W) // _NB


def _sc_body(chidx_hbm, widx_hbm, char_hbm, word_hbm,
             tgt_out, wctx_out,
             chidx_v, widx_v, chrows_v, wrows_v, tacc_v,
             hsem, wsem):
    wid = lax.axis_index("s") * _NC + lax.axis_index("c")
    nch = _BPW * _NCH
    nw = _BPW * _W

    pltpu.sync_copy(chidx_hbm.at[pl.ds(wid * nch, nch)], chidx_v)
    pltpu.sync_copy(widx_hbm.at[pl.ds(wid * nw, nw)], widx_v)

    def _scalars(ref, n):
        vals = [None] * n
        starts = sorted({*range(0, n - _L + 1, _L), n - _L})
        for s in starts:
            v = ref[pl.ds(s, _L)]
            for l in range(_L):
                if vals[s + l] is None:
                    vals[s + l] = v[l]
        return vals

    hids = _scalars(chidx_v, nch)
    wids = _scalars(widx_v, nw)

    hd = [pltpu.async_copy(char_hbm.at[pl.ds(hids[r], 1)],
                           chrows_v.at[pl.ds(r, 1)], hsem)
          for r in range(nch)]
    wd = [pltpu.async_copy(word_hbm.at[pl.ds(wids[r], 1)],
                           wrows_v.at[pl.ds(r, 1)], wsem)
          for r in range(nw)]

    offs = [k * _L for k in range(_NFULL)] + [_D - _L]

    for d in hd:
        d.wait()

    for b in range(_BPW):
        acc = [jnp.zeros((_L,), jnp.float32) for _ in range(len(offs))]
        for j in range(_NCH):
            r = b * _NCH + j
            m = jnp.where(hids[r] != 1, 1.0, 0.0)
            for k, o in enumerate(offs):
                acc[k] = acc[k] + chrows_v[r, pl.ds(o, _L)] * m
        for k, o in enumerate(offs):
            tacc_v[b, pl.ds(o, _L)] = acc[k]

    pltpu.sync_copy(tacc_v, tgt_out.at[pl.ds(wid * _BPW, _BPW)])

    for d in wd:
        d.wait()
    pltpu.sync_copy(wrows_v, wctx_out.at[pl.ds(wid * nw, nw)])


@functools.lru_cache(maxsize=1)
def _get_sc_gather():
    return functools.partial(
        pl.kernel,
        out_type=(jax.ShapeDtypeStruct((_B, _D), jnp.float32),
                  jax.ShapeDtypeStruct((_B * _W, _D), jnp.float32)),
        mesh=plsc.VectorSubcoreMesh(core_axis_name="c", subcore_axis_name="s"),
        scratch_types=[
            pltpu.VMEM((_BPW * _NCH,), jnp.int32),
            pltpu.VMEM((_BPW * _W,), jnp.int32),
            pltpu.VMEM((_BPW * _NCH, _D), jnp.float32),
            pltpu.VMEM((_BPW * _W, _D), jnp.float32),
            pltpu.VMEM((_BPW, _D), jnp.float32),
            pltpu.SemaphoreType.DMA,
            pltpu.SemaphoreType.DMA,
        ],
    )(_sc_body)


def _tc_body(cw_ref, cm_ref, tgtch_ref, wctx_ref, compot_ref, noise_ref,
             out_ref, tgtct_acc, tgt_acc):
    i = pl.program_id(0)

    @pl.when(i < _NA)
    def _phase_a():
        v0 = i * _VB
        iota_v = lax.broadcasted_iota(jnp.int32, (_VB, _B), 0) + v0
        cm = cm_ref[...]
        o = jnp.zeros((_VB, _B), jnp.float32)
        for j in range(_NCO):
            ids = cm[:, j][None, :]
            hit = (iota_v == ids) & (ids != 1)
            o = o + hit.astype(jnp.float32)
        blk = compot_ref[...]
        blk = jnp.where(jnp.isfinite(blk), blk, 0.0)
        part = lax.dot_general(blk, o, (((1,), (0,)), ((), ())),
                               preferred_element_type=jnp.float32)

        @pl.when(i == 0)
        def _():
            tgtct_acc[...] = part

        @pl.when(i > 0)
        def _():
            tgtct_acc[...] = tgtct_acc[...] + part

    @pl.when(i == _NA)
    def _start_b():
        tgt = tgtch_ref[...] + tgtct_acc[...].T
        tgt_acc[...] = tgt
        dfull = lax.dot_general(tgt, wctx_ref[...], (((1,), (1,)), ((), ())),
                                preferred_element_type=jnp.float32,
                                precision=lax.Precision.HIGHEST)
        row = lax.broadcasted_iota(jnp.int32, (_B, _B * _W), 0)
        col = lax.broadcasted_iota(jnp.int32, (_B, _B * _W), 1)
        bd = (col // _W) == row
        masked = jnp.where(bd, dfull, 0.0)
        gsel = ((lax.broadcasted_iota(jnp.int32, (_B * _W, _W), 0) % _W) ==
                lax.broadcasted_iota(jnp.int32, (_B * _W, _W), 1)
                ).astype(jnp.float32)
        dots = lax.dot_general(masked, gsel, (((1,), (0,)), ((), ())),
                               preferred_element_type=jnp.float32,
                               precision=lax.Precision.HIGHEST)
        mask = (cw_ref[...] != 1).astype(jnp.float32)
        dots = dots * mask
        sd = 1.0 / (1.0 + jnp.exp(-dots))
        out_ref[...] = jnp.sum(jnp.log(sd)).reshape(1, 1)

    @pl.when(i >= _NA)
    def _phase_b():
        nf = noise_ref[...].astype(jnp.float32)
        s = -lax.dot_general(tgt_acc[...], nf, (((1,), (1,)), ((), ())),
                             preferred_element_type=jnp.float32,
                             precision=lax.Precision.HIGHEST)
        sig = 1.0 / (1.0 + jnp.exp(-s))
        out_ref[...] = out_ref[...] + jnp.sum(jnp.log(sig + 1e-32))

    @pl.when(i == _NA + _NB - 1)
    def _fin():
        out_ref[...] = out_ref[...] * (-1.0 / _B)


_tc_loss = pl.pallas_call(
    _tc_body,
    grid=(_NA + _NB,),
    in_specs=[
        pl.BlockSpec((_B, _W), lambda i: (0, 0)),
        pl.BlockSpec((_B, _NCO), lambda i: (0, 0)),
        pl.BlockSpec((_B, _D), lambda i: (0, 0)),
        pl.BlockSpec((_B * _W, _D), lambda i: (0, 0)),
        pl.BlockSpec((_D, _VB), lambda i: (0, jnp.minimum(i, _NA - 1))),
        pl.BlockSpec((_NBLK, _D), lambda i: (jnp.maximum(i - _NA, 0), 0)),
    ],
    out_specs=pl.BlockSpec((1, 1), lambda i: (0, 0)),
    out_shape=jax.ShapeDtypeStruct((1, 1), jnp.float32),
    scratch_shapes=[
        pltpu.VMEM((_D, _B), jnp.float32),
        pltpu.VMEM((_B, _D), jnp.float32),
    ],
)


def kernel(tgt_chars, tgt_compos, ctx_words, noise, word_emb, char_emb,
           compo_emb):
    chidx = tgt_chars.reshape(-1).astype(jnp.int32)
    widx = ctx_words.reshape(-1).astype(jnp.int32)
    tgt_ch, wctx = _get_sc_gather()(chidx, widx, char_emb, word_emb)
    noise2 = noise.reshape(_B * _W, _D).astype(jnp.int32)
    compot = compo_emb.T
    loss2d = _tc_loss(ctx_words.astype(jnp.int32),
                      tgt_compos.astype(jnp.int32), tgt_ch, wctx, compot,
                      noise2)
    return loss2d[0, 0]
